# baseline (speedup 1.0000x reference)
import jax
import jax.numpy as jnp
from jax import lax
from jax.experimental import pallas as pl
from jax.experimental.pallas import tpu as pltpu

B, N, M, D = 4, 4096, 4096, 3
K = D + 2
RT = 512
T = N // RT


def _chamfer_body(xa_ref, ya_ref, out_ref, colmin_ref):
    b = pl.program_id(0)
    t = pl.program_id(1)

    xa = xa_ref[0]
    ya = ya_ref[0]

    d2 = lax.dot_general(
        xa, ya, (((1,), (0,)), ((), ())),
        preferred_element_type=jnp.float32)

    scale = 1.0 / (B * N)
    rowmin = jnp.min(d2, axis=1)
    rowsum = jnp.sum(jnp.maximum(rowmin, 0.0)) * scale

    @pl.when(jnp.logical_and(b == 0, t == 0))
    def _():
        out_ref[...] = jnp.zeros((1, 1), jnp.float32)

    out_ref[...] += rowsum

    colpart = jnp.min(d2, axis=0, keepdims=True)

    @pl.when(t == 0)
    def _():
        colmin_ref[...] = colpart

    @pl.when(t != 0)
    def _():
        colmin_ref[...] = jnp.minimum(colmin_ref[...], colpart)

    @pl.when(t == T - 1)
    def _():
        out_ref[...] += jnp.sum(
            jnp.maximum(colmin_ref[...], 0.0)) * scale


def kernel(x, y):
    x2 = jnp.sum(x * x, axis=-1, keepdims=True)
    y2 = jnp.sum(y * y, axis=-1, keepdims=True)
    ones_x = jnp.ones((B, N, 1), jnp.float32)
    ones_y = jnp.ones((B, M, 1), jnp.float32)
    xa = jnp.concatenate([-2.0 * x, ones_x, x2], axis=-1)
    ya = jnp.transpose(
        jnp.concatenate([y, y2, ones_y], axis=-1), (0, 2, 1))

    out = pl.pallas_call(
        _chamfer_body,
        grid=(B, T),
        in_specs=[
            pl.BlockSpec((1, RT, K), lambda b, t: (b, t, 0)),
            pl.BlockSpec((1, K, M), lambda b, t: (b, 0, 0)),
        ],
        out_specs=pl.BlockSpec((1, 1), lambda b, t: (0, 0)),
        out_shape=jax.ShapeDtypeStruct((1, 1), jnp.float32),
        scratch_shapes=[pltpu.VMEM((1, M), jnp.float32)],
        compiler_params=pltpu.CompilerParams(
            dimension_semantics=("arbitrary", "arbitrary")),
    )(xa, ya)
    return out[0, 0]


# speedup vs baseline: 1.0951x; 1.0951x over previous
import jax
import jax.numpy as jnp
from jax import lax
from jax.experimental import pallas as pl
from jax.experimental.pallas import tpu as pltpu

B, N, M, D = 4, 4096, 4096, 3
RT = 512
T = N // RT


def _chamfer_body(x_ref, yt_ref, out_ref, colmin_ref):
    b = pl.program_id(0)
    t = pl.program_id(1)

    xb = x_ref[0]
    ybt = yt_ref[0]

    nxy2 = lax.dot_general(
        xb * -2.0, ybt, (((1,), (0,)), ((), ())),
        preferred_element_type=jnp.float32)
    x2 = jnp.sum(xb * xb, axis=1)[:, None]
    y2 = jnp.sum(ybt * ybt, axis=0)[None, :]

    u = nxy2 + y2
    w = nxy2 + x2

    scale = 1.0 / (B * N)
    rowmin = jnp.min(u, axis=1)[:, None] + x2
    rowsum = jnp.sum(jnp.maximum(rowmin, 0.0)) * scale

    @pl.when(jnp.logical_and(b == 0, t == 0))
    def _():
        out_ref[...] = jnp.zeros((1, 1), jnp.float32)

    out_ref[...] += rowsum

    colpart = jnp.min(w, axis=0, keepdims=True)

    @pl.when(t == 0)
    def _():
        colmin_ref[...] = colpart

    @pl.when(t != 0)
    def _():
        colmin_ref[...] = jnp.minimum(colmin_ref[...], colpart)

    @pl.when(t == T - 1)
    def _():
        out_ref[...] += jnp.sum(
            jnp.maximum(colmin_ref[...] + y2, 0.0)) * scale


def kernel(x, y):
    yt = jnp.transpose(y, (0, 2, 1))
    out = pl.pallas_call(
        _chamfer_body,
        grid=(B, T),
        in_specs=[
            pl.BlockSpec((1, RT, D), lambda b, t: (b, t, 0)),
            pl.BlockSpec((1, D, M), lambda b, t: (b, 0, 0)),
        ],
        out_specs=pl.BlockSpec((1, 1), lambda b, t: (0, 0)),
        out_shape=jax.ShapeDtypeStruct((1, 1), jnp.float32),
        scratch_shapes=[pltpu.VMEM((1, M), jnp.float32)],
        compiler_params=pltpu.CompilerParams(
            dimension_semantics=("arbitrary", "arbitrary")),
    )(x, yt)
    return out[0, 0]


# speedup vs baseline: 1.1008x; 1.0052x over previous
import jax
import jax.numpy as jnp
from jax import lax
from jax.experimental import pallas as pl
from jax.experimental.pallas import tpu as pltpu

B, N, M, D = 4, 4096, 4096, 3
RT = 1024
T = N // RT


def _chamfer_body(x_ref, yt_ref, out_ref, colmin_ref):
    b = pl.program_id(0)
    t = pl.program_id(1)

    xb = x_ref[0]
    ybt = yt_ref[0]

    nxy2 = lax.dot_general(
        xb * -2.0, ybt, (((1,), (0,)), ((), ())),
        preferred_element_type=jnp.float32)
    x2 = jnp.sum(xb * xb, axis=1)[:, None]
    y2 = jnp.sum(ybt * ybt, axis=0)[None, :]

    u = nxy2 + y2
    w = nxy2 + x2

    scale = 1.0 / (B * N)
    rowmin = jnp.min(u, axis=1)[:, None] + x2
    rowsum = jnp.sum(jnp.maximum(rowmin, 0.0)) * scale

    @pl.when(jnp.logical_and(b == 0, t == 0))
    def _():
        out_ref[...] = jnp.zeros((1, 1), jnp.float32)

    out_ref[...] += rowsum

    colpart = jnp.min(w, axis=0, keepdims=True)

    @pl.when(t == 0)
    def _():
        colmin_ref[...] = colpart

    @pl.when(t != 0)
    def _():
        colmin_ref[...] = jnp.minimum(colmin_ref[...], colpart)

    @pl.when(t == T - 1)
    def _():
        out_ref[...] += jnp.sum(
            jnp.maximum(colmin_ref[...] + y2, 0.0)) * scale


def kernel(x, y):
    yt = jnp.transpose(y, (0, 2, 1))
    out = pl.pallas_call(
        _chamfer_body,
        grid=(B, T),
        in_specs=[
            pl.BlockSpec((1, RT, D), lambda b, t: (b, t, 0)),
            pl.BlockSpec((1, D, M), lambda b, t: (b, 0, 0)),
        ],
        out_specs=pl.BlockSpec((1, 1), lambda b, t: (0, 0)),
        out_shape=jax.ShapeDtypeStruct((1, 1), jnp.float32),
        scratch_shapes=[pltpu.VMEM((1, M), jnp.float32)],
        compiler_params=pltpu.CompilerParams(
            dimension_semantics=("arbitrary", "arbitrary")),
    )(x, yt)
    return out[0, 0]


# speedup vs baseline: 1.4692x; 1.3346x over previous
import jax
import jax.numpy as jnp
from jax import lax
from jax.experimental import pallas as pl
from jax.experimental.pallas import tpu as pltpu

B, N, M, D = 4, 4096, 4096, 3
RT = 1024
T = N // RT
NC = 4
CR = RT // NC
G = CR // 8


def _chamfer_body(x_ref, yt_ref, out_ref, colmin_ref):
    b = pl.program_id(0)
    t = pl.program_id(1)

    ybt = yt_ref[0]
    y2 = jnp.sum(ybt * ybt, axis=0)[None, :]
    scale = 1.0 / (B * N)

    @pl.when(jnp.logical_and(b == 0, t == 0))
    def _():
        out_ref[...] = jnp.zeros((1, 1), jnp.float32)

    @pl.when(t == 0)
    def _():
        colmin_ref[...] = jnp.full((8, M), jnp.inf, jnp.float32)

    rowtotal = jnp.zeros((), jnp.float32)
    colmin8 = colmin_ref[...]
    for c in range(NC):
        xc = x_ref[0, c * CR:(c + 1) * CR, :]
        nxy2 = lax.dot_general(
            xc * -2.0, ybt, (((1,), (0,)), ((), ())),
            preferred_element_type=jnp.float32)
        x2 = jnp.sum(xc * xc, axis=1)[:, None]

        u = nxy2 + y2
        rowmin = jnp.min(u, axis=1)[:, None] + x2
        rowtotal += jnp.sum(jnp.maximum(rowmin, 0.0))

        w = nxy2 + x2
        colmin8 = jnp.minimum(
            colmin8, jnp.min(w.reshape(G, 8, M), axis=0))

    colmin_ref[...] = colmin8
    out_ref[...] += rowtotal * scale

    @pl.when(t == T - 1)
    def _():
        cm = jnp.min(colmin_ref[...], axis=0)[None, :] + y2
        out_ref[...] += jnp.sum(jnp.maximum(cm, 0.0)) * scale


def kernel(x, y):
    yt = jnp.transpose(y, (0, 2, 1))
    out = pl.pallas_call(
        _chamfer_body,
        grid=(B, T),
        in_specs=[
            pl.BlockSpec((1, RT, D), lambda b, t: (b, t, 0)),
            pl.BlockSpec((1, D, M), lambda b, t: (b, 0, 0)),
        ],
        out_specs=pl.BlockSpec((1, 1), lambda b, t: (0, 0)),
        out_shape=jax.ShapeDtypeStruct((1, 1), jnp.float32),
        scratch_shapes=[pltpu.VMEM((8, M), jnp.float32)],
        compiler_params=pltpu.CompilerParams(
            dimension_semantics=("arbitrary", "arbitrary")),
    )(x, yt)
    return out[0, 0]


# speedup vs baseline: 1.5926x; 1.0840x over previous
import jax
import jax.numpy as jnp
from jax import lax
from jax.experimental import pallas as pl
from jax.experimental.pallas import tpu as pltpu

B, N, M, D = 4, 4096, 4096, 3
NC = 16
CR = N // NC
G = CR // 8


def _chamfer_body(x_ref, yt_ref, out_ref):
    b = pl.program_id(0)

    ybt = yt_ref[0]
    y2 = jnp.sum(ybt * ybt, axis=0)[None, :]
    scale = 1.0 / (B * N)

    @pl.when(b == 0)
    def _():
        out_ref[...] = jnp.zeros((1, 1), jnp.float32)

    rowtotal = jnp.zeros((), jnp.float32)
    colmin8 = jnp.full((8, M), jnp.inf, jnp.float32)
    for c in range(NC):
        xc = x_ref[0, c * CR:(c + 1) * CR, :]
        nxy2 = lax.dot_general(
            xc * -2.0, ybt, (((1,), (0,)), ((), ())),
            preferred_element_type=jnp.float32)
        x2 = jnp.sum(xc * xc, axis=1)[:, None]

        u = nxy2 + y2
        rowmin = jnp.min(u, axis=1)[:, None] + x2
        rowtotal += jnp.sum(jnp.maximum(rowmin, 0.0))

        w = nxy2 + x2
        colmin8 = jnp.minimum(
            colmin8, jnp.min(w.reshape(G, 8, M), axis=0))

    cm = jnp.min(colmin8, axis=0)[None, :] + y2
    coltotal = jnp.sum(jnp.maximum(cm, 0.0))
    out_ref[...] += (rowtotal + coltotal) * scale


def kernel(x, y):
    yt = jnp.transpose(y, (0, 2, 1))
    out = pl.pallas_call(
        _chamfer_body,
        grid=(B,),
        in_specs=[
            pl.BlockSpec((1, N, D), lambda b: (b, 0, 0)),
            pl.BlockSpec((1, D, M), lambda b: (b, 0, 0)),
        ],
        out_specs=pl.BlockSpec((1, 1), lambda b: (0, 0)),
        out_shape=jax.ShapeDtypeStruct((1, 1), jnp.float32),
        compiler_params=pltpu.CompilerParams(
            dimension_semantics=("arbitrary",)),
    )(x, yt)
    return out[0, 0]
